# baseline (speedup 1.0000x reference)
import functools

import jax
import jax.numpy as jnp
from jax import lax
from jax.experimental import pallas as pl
from jax.experimental.pallas import tpu as pltpu
from jax.experimental.pallas import tpu_sc as plsc

HIDDEN = 128
LANES = 16
NJ = HIDDEN // LANES
EPS = 1e-5
NUM_CORES = 2
NUM_SUBCORES = 16
NUM_WORKERS = NUM_CORES * NUM_SUBCORES
CHUNK = 256
IVEC = 128
NSPLIT = CHUNK // IVEC


@functools.lru_cache(maxsize=None)
def _build(n_tokens: int, small_rows: int):
  n_per_w = n_tokens // NUM_WORKERS
  n_chunks = n_per_w // CHUNK
  assert n_tokens % NUM_WORKERS == 0 and n_per_w % CHUNK == 0
  assert n_chunks % 2 == 0

  mesh = plsc.VectorSubcoreMesh(
      core_axis_name="c", subcore_axis_name="s",
      num_cores=NUM_CORES, num_subcores=NUM_SUBCORES)

  @functools.partial(
      pl.kernel,
      mesh=mesh,
      compiler_params=pltpu.CompilerParams(needs_layout_passes=False),
      out_type=jax.ShapeDtypeStruct((n_tokens, HIDDEN), jnp.float32),
      scratch_types=[
          pltpu.VMEM((3, NSPLIT, IVEC), jnp.int32),
          pltpu.VMEM((3, NSPLIT, IVEC), jnp.int32),
          pltpu.VMEM((CHUNK, HIDDEN), jnp.float32),
          pltpu.VMEM((CHUNK, HIDDEN), jnp.float32),
          pltpu.VMEM((2 * HIDDEN,), jnp.float32),
          pltpu.VMEM_SHARED((small_rows, HIDDEN), jnp.float32),
          pltpu.SemaphoreType.DMA,
          pltpu.SemaphoreType.DMA,
          pltpu.SemaphoreType.DMA,
          pltpu.SemaphoreType.DMA,
          pltpu.SemaphoreType.DMA,
          pltpu.SemaphoreType.DMA,
          pltpu.SemaphoreType.DMA,
          pltpu.SemaphoreType.DMA,
      ],
  )
  def k(idx3_hbm, word_hbm, small_hbm, gb_hbm, out_hbm,
        idx0_v, idx1_v, rows0_v, rows1_v, gb_v, small_sh,
        isem0, isem1, gsem0, gsem1, psem0, psem1, wsem0, wsem1):
    wid = lax.axis_index("s") * NUM_CORES + lax.axis_index("c")
    base_w = wid * n_per_w
    blk_w = wid * n_chunks

    @pl.when(lax.axis_index("s") == 0)
    def _():
      pltpu.sync_copy(small_hbm, small_sh)

    pltpu.sync_copy(gb_hbm, gb_v)
    plsc.subcore_barrier()

    iota = lax.iota(jnp.int32, 16)
    perms = [jnp.bitwise_xor(iota, jnp.int32(1 << kk)) for kk in range(4)]
    gammas = [gb_v[pl.ds(j * LANES, LANES)] for j in range(NJ)]
    betas = [gb_v[pl.ds(HIDDEN + j * LANES, LANES)] for j in range(NJ)]

    def compute(rows_v, lo, hi):
      @plsc.parallel_loop(lo, hi)
      def _(r):
        if True:
          xs = []
          sum_v = None
          sumsq_v = None
          for j in range(NJ):
            x = rows_v[r, pl.ds(j * LANES, LANES)]
            xs.append(x)
            sum_v = x if sum_v is None else sum_v + x
            sumsq_v = x * x if sumsq_v is None else sumsq_v + x * x
          for perm in perms:
            sum_v = sum_v + sum_v.at[perm].get(mode="promise_in_bounds")
            sumsq_v = sumsq_v + sumsq_v.at[perm].get(
                mode="promise_in_bounds")
          mean_v = sum_v * (1.0 / HIDDEN)
          var_v = sumsq_v * (1.0 / HIDDEN) - mean_v * mean_v
          v_v = var_v + EPS
          yi = jnp.int32(0x5F3759DF) - (plsc.bitcast(v_v, jnp.int32) >> 1)
          y = plsc.bitcast(yi, jnp.float32)
          half_v = v_v * 0.5
          for _ in range(2):
            y = y * (1.5 - half_v * y * y)
          for j in range(NJ):
            xh = (xs[j] - mean_v) * y
            rows_v[r, pl.ds(j * LANES, LANES)] = xh * gammas[j] + betas[j]

    def row_half(rows_v, h):
      return rows_v.at[pl.ds(h * IVEC, IVEC)]

    def start_word(idx_v, rows_v, gsem):
      for h in range(NSPLIT):
        pltpu.async_copy(word_hbm.at[idx_v.at[0, h]], row_half(rows_v, h),
                         gsem)

    def wait_word(idx_v, rows_v, gsem):
      for h in range(NSPLIT):
        pltpu.make_async_copy(word_hbm.at[idx_v.at[0, h]],
                              row_half(rows_v, h), gsem).wait()

    def start_addg(idx_v, rows_v, psem):
      for h in range(NSPLIT):
        pltpu.async_copy(small_sh.at[idx_v.at[1, h]], row_half(rows_v, h),
                         psem, add=True)
        pltpu.async_copy(small_sh.at[idx_v.at[2, h]], row_half(rows_v, h),
                         psem, add=True)

    def wait_addg(idx_v, rows_v, psem):
      for h in range(NSPLIT):
        pltpu.make_async_copy(small_sh.at[idx_v.at[1, h]],
                              row_half(rows_v, h), psem).wait()
        pltpu.make_async_copy(small_sh.at[idx_v.at[2, h]],
                              row_half(rows_v, h), psem).wait()

    def phase(ci, idx_cur, idx_nxt, rows_cur, rows_nxt, isem_cur, isem_nxt,
              gsem_nxt, psem_cur, psem_nxt, wsem_cur, wsem_nxt):
      base = base_w + ci * CHUNK

      wait_addg(idx_cur, rows_cur, psem_cur)

      @pl.when(ci < n_chunks - 2)
      def _():
        pltpu.async_copy(idx3_hbm.at[blk_w + ci + 2], idx_cur, isem_cur)

      @pl.when(ci > 0)
      def _():
        pltpu.make_async_copy(
            rows_nxt, out_hbm.at[pl.ds(base, CHUNK)], wsem_nxt).wait()

      @pl.when(ci < n_chunks - 1)
      def _():
        pltpu.make_async_copy(
            idx3_hbm.at[blk_w + ci + 1], idx_nxt, isem_nxt).wait()
        start_word(idx_nxt, rows_nxt, gsem_nxt)

      compute(rows_cur, 0, CHUNK // 2)

      @pl.when(ci < n_chunks - 1)
      def _():
        wait_word(idx_nxt, rows_nxt, gsem_nxt)
        start_addg(idx_nxt, rows_nxt, psem_nxt)

      compute(rows_cur, CHUNK // 2, CHUNK)
      pltpu.async_copy(rows_cur, out_hbm.at[pl.ds(base, CHUNK)], wsem_cur)

    pltpu.sync_copy(idx3_hbm.at[blk_w], idx0_v)
    pltpu.async_copy(idx3_hbm.at[blk_w + 1], idx1_v, isem1)
    start_word(idx0_v, rows0_v, gsem0)
    wait_word(idx0_v, rows0_v, gsem0)
    start_addg(idx0_v, rows0_v, psem0)

    def loop_body(i, c):
      ci = i * 2
      phase(ci, idx0_v, idx1_v, rows0_v, rows1_v,
            isem0, isem1, gsem1, psem0, psem1, wsem0, wsem1)
      phase(ci + 1, idx1_v, idx0_v, rows1_v, rows0_v,
            isem1, isem0, gsem0, psem1, psem0, wsem1, wsem0)
      return c

    lax.fori_loop(0, n_chunks // 2, loop_body, 0)
    last_base = base_w + (n_chunks - 1) * CHUNK
    pltpu.make_async_copy(
        rows1_v, out_hbm.at[pl.ds(last_base, CHUNK)], wsem1).wait()

  return k


def kernel(input_ids, position_ids, segment_ids, age_ids, word_table,
           pos_table, seg_table, age_table, ln_gamma, ln_beta):
  b, s = input_ids.shape
  n_tokens = b * s
  n_blocks = n_tokens // CHUNK
  pos_rows = pos_table.shape[0]
  age_rows = age_table.shape[0]
  age_aug = jnp.concatenate(
      [age_table + seg_table[0], age_table + seg_table[1]], axis=0)
  comb = (age_ids + age_rows * segment_ids + pos_rows)
  idx3 = jnp.stack([
      input_ids.reshape(n_blocks, NSPLIT, IVEC),
      position_ids.reshape(n_blocks, NSPLIT, IVEC),
      comb.reshape(n_blocks, NSPLIT, IVEC),
  ], axis=1).astype(jnp.int32)
  small = jnp.concatenate([pos_table, age_aug], axis=0)
  gb = jnp.concatenate([ln_gamma, ln_beta], axis=0)
  small_rows = pos_rows + 2 * age_rows
  k = _build(n_tokens, small_rows)
  out = k(idx3, word_table, small, gb)
  return out.reshape(b, s, HIDDEN)


# speedup vs baseline: 1.0105x; 1.0105x over previous
import functools

import jax
import jax.numpy as jnp
from jax import lax
from jax.experimental import pallas as pl
from jax.experimental.pallas import tpu as pltpu
from jax.experimental.pallas import tpu_sc as plsc

HIDDEN = 128
LANES = 16
NJ = HIDDEN // LANES
EPS = 1e-5
NUM_CORES = 2
NUM_SUBCORES = 16
NUM_WORKERS = NUM_CORES * NUM_SUBCORES
CHUNK = 256
IVEC = 128
NSPLIT = CHUNK // IVEC


@functools.lru_cache(maxsize=None)
def _build(n_tokens: int, small_rows: int):
  n_per_w = n_tokens // NUM_WORKERS
  n_chunks = n_per_w // CHUNK
  assert n_tokens % NUM_WORKERS == 0 and n_per_w % CHUNK == 0
  assert n_chunks % 2 == 0

  mesh = plsc.VectorSubcoreMesh(
      core_axis_name="c", subcore_axis_name="s",
      num_cores=NUM_CORES, num_subcores=NUM_SUBCORES)

  @functools.partial(
      pl.kernel,
      mesh=mesh,
      compiler_params=pltpu.CompilerParams(needs_layout_passes=False),
      out_type=jax.ShapeDtypeStruct((n_tokens, HIDDEN), jnp.float32),
      scratch_types=[
          pltpu.VMEM((3, NSPLIT, IVEC), jnp.int32),
          pltpu.VMEM((3, NSPLIT, IVEC), jnp.int32),
          pltpu.VMEM((CHUNK, HIDDEN), jnp.float32),
          pltpu.VMEM((CHUNK, HIDDEN), jnp.float32),
          pltpu.VMEM((2 * HIDDEN,), jnp.float32),
          pltpu.VMEM_SHARED((small_rows, HIDDEN), jnp.float32),
          pltpu.SemaphoreType.DMA,
          pltpu.SemaphoreType.DMA,
          pltpu.SemaphoreType.DMA,
          pltpu.SemaphoreType.DMA,
          pltpu.SemaphoreType.DMA,
          pltpu.SemaphoreType.DMA,
          pltpu.SemaphoreType.DMA,
          pltpu.SemaphoreType.DMA,
      ],
  )
  def k(idx3_hbm, word_hbm, small_hbm, gb_hbm, out_hbm,
        idx0_v, idx1_v, rows0_v, rows1_v, gb_v, small_sh,
        isem0, isem1, gsem0, gsem1, psem0, psem1, wsem0, wsem1):
    wid = lax.axis_index("s") * NUM_CORES + lax.axis_index("c")
    base_w = wid * n_per_w
    blk_w = wid * n_chunks

    @pl.when(lax.axis_index("s") == 0)
    def _():
      pltpu.sync_copy(small_hbm, small_sh)

    pltpu.sync_copy(gb_hbm, gb_v)
    plsc.subcore_barrier()

    iota = lax.iota(jnp.int32, 16)
    perms = [jnp.bitwise_xor(iota, jnp.int32(1 << kk)) for kk in range(4)]
    gammas = [gb_v[pl.ds(j * LANES, LANES)] for j in range(NJ)]
    betas = [gb_v[pl.ds(HIDDEN + j * LANES, LANES)] for j in range(NJ)]

    def compute(rows_v, lo, hi):
      @plsc.parallel_loop(lo, hi)
      def _(r):
        if True:
          xs = [rows_v[r, pl.ds(j * LANES, LANES)] for j in range(NJ)]
          def tree(vs):
            while len(vs) > 1:
              vs = [a + b for a, b in zip(vs[0::2], vs[1::2])]
            return vs[0]

          sum_v = tree(list(xs))
          sumsq_v = tree([x * x for x in xs])
          for perm in perms:
            sum_v = sum_v + sum_v.at[perm].get(mode="promise_in_bounds")
            sumsq_v = sumsq_v + sumsq_v.at[perm].get(
                mode="promise_in_bounds")
          mean_v = sum_v * (1.0 / HIDDEN)
          var_v = sumsq_v * (1.0 / HIDDEN) - mean_v * mean_v
          v_v = var_v + EPS
          yi = jnp.int32(0x5F3759DF) - (plsc.bitcast(v_v, jnp.int32) >> 1)
          y = plsc.bitcast(yi, jnp.float32)
          half_v = v_v * 0.5
          for _ in range(1):
            y = y * (1.5 - half_v * y * y)
          for j in range(NJ):
            xh = (xs[j] - mean_v) * y
            rows_v[r, pl.ds(j * LANES, LANES)] = xh * gammas[j] + betas[j]

    def row_half(rows_v, h):
      return rows_v.at[pl.ds(h * IVEC, IVEC)]

    def start_word(idx_v, rows_v, gsem):
      for h in range(NSPLIT):
        pltpu.async_copy(word_hbm.at[idx_v.at[0, h]], row_half(rows_v, h),
                         gsem)

    def wait_word(idx_v, rows_v, gsem):
      for h in range(NSPLIT):
        pltpu.make_async_copy(word_hbm.at[idx_v.at[0, h]],
                              row_half(rows_v, h), gsem).wait()

    def start_addg(idx_v, rows_v, psem):
      for h in range(NSPLIT):
        pltpu.async_copy(small_sh.at[idx_v.at[1, h]], row_half(rows_v, h),
                         psem, add=True)
        pltpu.async_copy(small_sh.at[idx_v.at[2, h]], row_half(rows_v, h),
                         psem, add=True)

    def wait_addg(idx_v, rows_v, psem):
      for h in range(NSPLIT):
        pltpu.make_async_copy(small_sh.at[idx_v.at[1, h]],
                              row_half(rows_v, h), psem).wait()
        pltpu.make_async_copy(small_sh.at[idx_v.at[2, h]],
                              row_half(rows_v, h), psem).wait()

    def phase(ci, idx_cur, idx_nxt, rows_cur, rows_nxt, isem_cur, isem_nxt,
              gsem_nxt, psem_cur, psem_nxt, wsem_cur, wsem_nxt):
      base = base_w + ci * CHUNK

      wait_addg(idx_cur, rows_cur, psem_cur)

      @pl.when(ci < n_chunks - 2)
      def _():
        pltpu.async_copy(idx3_hbm.at[blk_w + ci + 2], idx_cur, isem_cur)

      @pl.when(ci > 0)
      def _():
        pltpu.make_async_copy(
            rows_nxt, out_hbm.at[pl.ds(base, CHUNK)], wsem_nxt).wait()

      @pl.when(ci < n_chunks - 1)
      def _():
        pltpu.make_async_copy(
            idx3_hbm.at[blk_w + ci + 1], idx_nxt, isem_nxt).wait()
        start_word(idx_nxt, rows_nxt, gsem_nxt)

      compute(rows_cur, 0, CHUNK // 2)

      @pl.when(ci < n_chunks - 1)
      def _():
        wait_word(idx_nxt, rows_nxt, gsem_nxt)
        start_addg(idx_nxt, rows_nxt, psem_nxt)

      compute(rows_cur, CHUNK // 2, CHUNK)
      pltpu.async_copy(rows_cur, out_hbm.at[pl.ds(base, CHUNK)], wsem_cur)

    pltpu.sync_copy(idx3_hbm.at[blk_w], idx0_v)
    pltpu.async_copy(idx3_hbm.at[blk_w + 1], idx1_v, isem1)
    start_word(idx0_v, rows0_v, gsem0)
    wait_word(idx0_v, rows0_v, gsem0)
    start_addg(idx0_v, rows0_v, psem0)

    def loop_body(i, c):
      ci = i * 2
      phase(ci, idx0_v, idx1_v, rows0_v, rows1_v,
            isem0, isem1, gsem1, psem0, psem1, wsem0, wsem1)
      phase(ci + 1, idx1_v, idx0_v, rows1_v, rows0_v,
            isem1, isem0, gsem0, psem1, psem0, wsem1, wsem0)
      return c

    lax.fori_loop(0, n_chunks // 2, loop_body, 0)
    last_base = base_w + (n_chunks - 1) * CHUNK
    pltpu.make_async_copy(
        rows1_v, out_hbm.at[pl.ds(last_base, CHUNK)], wsem1).wait()

  return k


def kernel(input_ids, position_ids, segment_ids, age_ids, word_table,
           pos_table, seg_table, age_table, ln_gamma, ln_beta):
  b, s = input_ids.shape
  n_tokens = b * s
  n_blocks = n_tokens // CHUNK
  pos_rows = pos_table.shape[0]
  age_rows = age_table.shape[0]
  age_aug = jnp.concatenate(
      [age_table + seg_table[0], age_table + seg_table[1]], axis=0)
  comb = (age_ids + age_rows * segment_ids + pos_rows)
  idx3 = jnp.stack([
      input_ids.reshape(n_blocks, NSPLIT, IVEC),
      position_ids.reshape(n_blocks, NSPLIT, IVEC),
      comb.reshape(n_blocks, NSPLIT, IVEC),
  ], axis=1).astype(jnp.int32)
  small = jnp.concatenate([pos_table, age_aug], axis=0)
  gb = jnp.concatenate([ln_gamma, ln_beta], axis=0)
  small_rows = pos_rows + 2 * age_rows
  k = _build(n_tokens, small_rows)
  out = k(idx3, word_table, small, gb)
  return out.reshape(b, s, HIDDEN)
